# Initial kernel scaffold; baseline (speedup 1.0000x reference)
import jax
import jax.numpy as jnp
from jax.experimental import pallas as pl


def kernel(x, u, edge_index, W_w, W_b, phi_w, phi_b, head_w, head_b):
    raise NotImplementedError("write your pallas kernel here")


# speedup vs baseline: 6.3322x; 6.3322x over previous
import functools

import jax
import jax.numpy as jnp
from jax import lax
from jax.experimental import pallas as pl
from jax.experimental.pallas import tpu as pltpu
from jax.experimental.pallas import tpu_sc as plsc

N_NODES = 10000
N_EDGES = 320000
D = 128

CHUNK = 128
NUM_CHUNKS = N_EDGES // CHUNK
NC, NS = 2, 16
NW = NC * NS
BASE_CHUNKS = NUM_CHUNKS // NW
EXTRA = NUM_CHUNKS - BASE_CHUNKS * NW
ROWS_PER_SUB = 624
TAIL_ROWS = N_NODES - NS * ROWS_PER_SUB
ZROWS = 16

ROW_BLK = 1000


def _h_body(x_ref, w_ref, b_ref, o_ref):
    o_ref[...] = (
        jnp.dot(x_ref[...], w_ref[...], preferred_element_type=jnp.float32)
        + b_ref[...]
    )


def _h_matmul(x, w_t, b2d):
    return pl.pallas_call(
        _h_body,
        grid=(N_NODES // ROW_BLK,),
        in_specs=[
            pl.BlockSpec((ROW_BLK, D), lambda i: (i, 0)),
            pl.BlockSpec((D, D), lambda i: (0, 0)),
            pl.BlockSpec((1, D), lambda i: (0, 0)),
        ],
        out_specs=pl.BlockSpec((ROW_BLK, D), lambda i: (i, 0)),
        out_shape=jax.ShapeDtypeStruct((N_NODES, D), jnp.float32),
    )(x, w_t, b2d)


def _sc_body(src_hbm, dst_hbm, h_hbm, out0, out1,
             zbuf, sidx, didx, rows, agg, sem):
    c = lax.axis_index("c")
    s = lax.axis_index("s")
    wid = s * NC + c

    zero = jnp.zeros((16,), jnp.float32)

    def zrow(r, carry):
        for cc in range(8):
            zbuf[r, pl.ds(cc * 16, 16)] = zero
        return carry

    lax.fori_loop(0, ZROWS, zrow, 0)
    rs = pl.multiple_of(s * ROWS_PER_SUB, 8)

    def zcopy(b, carry):
        off = pl.multiple_of(rs + b * ZROWS, 8)
        pltpu.sync_copy(zbuf, agg.at[pl.ds(off, ZROWS)])
        return carry

    lax.fori_loop(0, ROWS_PER_SUB // ZROWS, zcopy, 0)

    @pl.when(s == NS - 1)
    def _():
        pltpu.sync_copy(zbuf, agg.at[pl.ds(NS * ROWS_PER_SUB, TAIL_ROWS)])

    plsc.subcore_barrier()

    start = wid * BASE_CHUNKS + jnp.minimum(wid, EXTRA)
    cnt = BASE_CHUNKS + jnp.where(wid < EXTRA, 1, 0)

    def chunk_body(i, carry):
        off = pl.multiple_of((start + i) * CHUNK, CHUNK)
        pltpu.sync_copy(src_hbm.at[pl.ds(off, CHUNK)], sidx)
        pltpu.sync_copy(dst_hbm.at[pl.ds(off, CHUNK)], didx)
        pltpu.async_copy(h_hbm.at[sidx], rows, sem).wait()
        pltpu.sync_copy(rows, agg.at[didx], add=True)
        return carry

    lax.fori_loop(0, cnt, chunk_body, 0)
    plsc.subcore_barrier()

    out = [out0, out1]
    for ci in range(NC):
        @pl.when(c == ci)
        def _(ci=ci):
            pltpu.sync_copy(agg.at[pl.ds(rs, ROWS_PER_SUB)],
                            out[ci].at[pl.ds(rs, ROWS_PER_SUB)])

            @pl.when(s == NS - 1)
            def _():
                pltpu.sync_copy(
                    agg.at[pl.ds(NS * ROWS_PER_SUB, TAIL_ROWS)],
                    out[ci].at[pl.ds(NS * ROWS_PER_SUB, TAIL_ROWS)])


@functools.partial(jax.jit, donate_argnums=())
def _segment_sum_sc(src, dst, h):
    mesh = plsc.VectorSubcoreMesh(core_axis_name="c", subcore_axis_name="s")
    f = pl.kernel(
        _sc_body,
        out_type=(
            jax.ShapeDtypeStruct((N_NODES, D), jnp.float32),
            jax.ShapeDtypeStruct((N_NODES, D), jnp.float32),
        ),
        mesh=mesh,
        scratch_types=[
            pltpu.VMEM((ZROWS, D), jnp.float32),
            pltpu.VMEM((CHUNK,), jnp.int32),
            pltpu.VMEM((CHUNK,), jnp.int32),
            pltpu.VMEM((CHUNK, D), jnp.float32),
            pltpu.VMEM_SHARED((N_NODES, D), jnp.float32),
            pltpu.SemaphoreType.DMA,
        ],
    )
    return f(src, dst, h)


def _out_body(p0_ref, p1_ref, u_ref, phit_ref, phib_ref, hwt_ref, hb_ref,
              h2_ref, y_ref):
    t = (
        p0_ref[...] + p1_ref[...]
        + jnp.dot(u_ref[...], phit_ref[...], preferred_element_type=jnp.float32)
        + phib_ref[...]
    )
    h2 = jnp.maximum(t, 0.0)
    h2_ref[...] = h2
    y_ref[...] = (
        jnp.dot(h2, hwt_ref[...], preferred_element_type=jnp.float32)
        + hb_ref[...]
    )


def _out_tc(p0, p1, u, phi_t, phib2d, head_t, headb2d):
    return pl.pallas_call(
        _out_body,
        grid=(N_NODES // ROW_BLK,),
        in_specs=[
            pl.BlockSpec((ROW_BLK, D), lambda i: (i, 0)),
            pl.BlockSpec((ROW_BLK, D), lambda i: (i, 0)),
            pl.BlockSpec((ROW_BLK, D), lambda i: (i, 0)),
            pl.BlockSpec((D, D), lambda i: (0, 0)),
            pl.BlockSpec((1, D), lambda i: (0, 0)),
            pl.BlockSpec((D, 64), lambda i: (0, 0)),
            pl.BlockSpec((1, 64), lambda i: (0, 0)),
        ],
        out_specs=[
            pl.BlockSpec((ROW_BLK, D), lambda i: (i, 0)),
            pl.BlockSpec((ROW_BLK, 64), lambda i: (i, 0)),
        ],
        out_shape=[
            jax.ShapeDtypeStruct((N_NODES, D), jnp.float32),
            jax.ShapeDtypeStruct((N_NODES, 64), jnp.float32),
        ],
    )(p0, p1, u, phi_t, phib2d, head_t, headb2d)


def kernel(x, u, edge_index, W_w, W_b, phi_w, phi_b, head_w, head_b):
    h = _h_matmul(x, W_w.T, W_b.reshape(1, D))
    p0, p1 = _segment_sum_sc(edge_index[0], edge_index[1], h)
    h2, y = _out_tc(p0, p1, u, phi_w.T, phi_b.reshape(1, D),
                    head_w.T, head_b.reshape(1, 64))
    return (h2, y)


# speedup vs baseline: 10.2704x; 1.6219x over previous
import functools

import jax
import jax.numpy as jnp
from jax import lax
from jax.experimental import pallas as pl
from jax.experimental.pallas import tpu as pltpu
from jax.experimental.pallas import tpu_sc as plsc

N_NODES = 10000
N_EDGES = 320000
D = 128

CHUNK = 128
NUM_CHUNKS = N_EDGES // CHUNK
NC, NS = 2, 16
NW = NC * NS
BASE_CHUNKS = NUM_CHUNKS // NW
EXTRA = NUM_CHUNKS - BASE_CHUNKS * NW
ROWS_PER_SUB = 624
TAIL_ROWS = N_NODES - NS * ROWS_PER_SUB
ZROWS = 16

ROW_BLK = 1000


def _h_body(x_ref, w_ref, b_ref, o_ref):
    o_ref[...] = (
        jnp.dot(x_ref[...], w_ref[...], preferred_element_type=jnp.float32)
        + b_ref[...]
    )


def _h_matmul(x, w_t, b2d):
    return pl.pallas_call(
        _h_body,
        grid=(N_NODES // ROW_BLK,),
        in_specs=[
            pl.BlockSpec((ROW_BLK, D), lambda i: (i, 0)),
            pl.BlockSpec((D, D), lambda i: (0, 0)),
            pl.BlockSpec((1, D), lambda i: (0, 0)),
        ],
        out_specs=pl.BlockSpec((ROW_BLK, D), lambda i: (i, 0)),
        out_shape=jax.ShapeDtypeStruct((N_NODES, D), jnp.float32),
    )(x, w_t, b2d)


def _sc_body(src_hbm, dst_hbm, h_hbm, out0, out1,
             zbuf, sidx_all, didx0, didx1, rows0, rows1, agg,
             dsem0, dsem1, gsem0, gsem1):
    c = lax.axis_index("c")
    s = lax.axis_index("s")
    wid = s * NC + c

    zero = jnp.zeros((16,), jnp.float32)

    def zrow(r, carry):
        for cc in range(8):
            zbuf[r, pl.ds(cc * 16, 16)] = zero
        return carry

    lax.fori_loop(0, ZROWS, zrow, 0)
    rs = pl.multiple_of(s * ROWS_PER_SUB, 8)

    def zcopy(b, carry):
        off = pl.multiple_of(rs + b * ZROWS, 8)
        pltpu.sync_copy(zbuf, agg.at[pl.ds(off, ZROWS)])
        return carry

    lax.fori_loop(0, ROWS_PER_SUB // ZROWS, zcopy, 0)

    @pl.when(s == NS - 1)
    def _():
        pltpu.sync_copy(zbuf, agg.at[pl.ds(NS * ROWS_PER_SUB, TAIL_ROWS)])

    plsc.subcore_barrier()

    start = wid * BASE_CHUNKS + jnp.minimum(wid, EXTRA)
    has_extra = wid < EXTRA

    didx = [didx0, didx1]
    rows = [rows0, rows1]
    dsem = [dsem0, dsem1]
    gsem = [gsem0, gsem1]

    eoff = pl.multiple_of(start * CHUNK, CHUNK)
    pltpu.sync_copy(src_hbm.at[pl.ds(eoff, (BASE_CHUNKS + 1) * CHUNK)],
                    sidx_all)

    def chunk_off(j):
        cid = jnp.minimum(start + j, NUM_CHUNKS - 1)
        return pl.multiple_of(cid * CHUNK, CHUNK)

    def fire_d(j, b):
        pltpu.async_copy(dst_hbm.at[pl.ds(chunk_off(j), CHUNK)], didx[b],
                         dsem[b])

    def wait_d(b):
        pltpu.make_async_copy(dst_hbm.at[pl.ds(0, CHUNK)], didx[b],
                              dsem[b]).wait()

    def fire_g(j, b):
        joff = pl.multiple_of(jnp.minimum(j, BASE_CHUNKS) * CHUNK, CHUNK)
        pltpu.async_copy(h_hbm.at[sidx_all.at[pl.ds(joff, CHUNK)]], rows[b],
                         gsem[b])

    def wait_g(b):
        pltpu.make_async_copy(h_hbm.at[sidx_all.at[pl.ds(0, CHUNK)]], rows[b],
                              gsem[b]).wait()

    fire_d(0, 0)
    fire_d(1, 1)
    fire_g(0, 0)

    def pipe_body(i, carry):
        j0 = 2 * i
        wait_g(0)
        fire_g(j0 + 1, 1)
        wait_d(0)
        pltpu.sync_copy(rows[0], agg.at[didx[0]], add=True)
        fire_d(j0 + 2, 0)
        wait_g(1)
        fire_g(j0 + 2, 0)
        wait_d(1)
        pltpu.sync_copy(rows[1], agg.at[didx[1]], add=True)
        fire_d(j0 + 3, 1)
        return carry

    lax.fori_loop(0, BASE_CHUNKS // 2, pipe_body, 0)

    wait_g(0)
    wait_d(0)
    wait_d(1)

    @pl.when(has_extra)
    def _():
        pltpu.sync_copy(rows[0], agg.at[didx[0]], add=True)

    plsc.subcore_barrier()

    out = [out0, out1]
    for ci in range(NC):
        @pl.when(c == ci)
        def _(ci=ci):
            pltpu.sync_copy(agg.at[pl.ds(rs, ROWS_PER_SUB)],
                            out[ci].at[pl.ds(rs, ROWS_PER_SUB)])

            @pl.when(s == NS - 1)
            def _():
                pltpu.sync_copy(
                    agg.at[pl.ds(NS * ROWS_PER_SUB, TAIL_ROWS)],
                    out[ci].at[pl.ds(NS * ROWS_PER_SUB, TAIL_ROWS)])


@functools.partial(jax.jit, donate_argnums=())
def _segment_sum_sc(src, dst, h):
    mesh = plsc.VectorSubcoreMesh(core_axis_name="c", subcore_axis_name="s")
    f = pl.kernel(
        _sc_body,
        out_type=(
            jax.ShapeDtypeStruct((N_NODES, D), jnp.float32),
            jax.ShapeDtypeStruct((N_NODES, D), jnp.float32),
        ),
        mesh=mesh,
        scratch_types=[
            pltpu.VMEM((ZROWS, D), jnp.float32),
            pltpu.VMEM(((BASE_CHUNKS + 1) * CHUNK,), jnp.int32),
            pltpu.VMEM((CHUNK,), jnp.int32),
            pltpu.VMEM((CHUNK,), jnp.int32),
            pltpu.VMEM((CHUNK, D), jnp.float32),
            pltpu.VMEM((CHUNK, D), jnp.float32),
            pltpu.VMEM_SHARED((N_NODES, D), jnp.float32),
            pltpu.SemaphoreType.DMA,
            pltpu.SemaphoreType.DMA,
            pltpu.SemaphoreType.DMA,
            pltpu.SemaphoreType.DMA,
        ],
    )
    src_pad = jnp.concatenate([src, src[:CHUNK]])
    return f(src_pad, dst, h)


def _out_body(p0_ref, p1_ref, u_ref, phit_ref, phib_ref, hwt_ref, hb_ref,
              h2_ref, y_ref):
    t = (
        p0_ref[...] + p1_ref[...]
        + jnp.dot(u_ref[...], phit_ref[...], preferred_element_type=jnp.float32)
        + phib_ref[...]
    )
    h2 = jnp.maximum(t, 0.0)
    h2_ref[...] = h2
    y_ref[...] = (
        jnp.dot(h2, hwt_ref[...], preferred_element_type=jnp.float32)
        + hb_ref[...]
    )


def _out_tc(p0, p1, u, phi_t, phib2d, head_t, headb2d):
    return pl.pallas_call(
        _out_body,
        grid=(N_NODES // ROW_BLK,),
        in_specs=[
            pl.BlockSpec((ROW_BLK, D), lambda i: (i, 0)),
            pl.BlockSpec((ROW_BLK, D), lambda i: (i, 0)),
            pl.BlockSpec((ROW_BLK, D), lambda i: (i, 0)),
            pl.BlockSpec((D, D), lambda i: (0, 0)),
            pl.BlockSpec((1, D), lambda i: (0, 0)),
            pl.BlockSpec((D, 64), lambda i: (0, 0)),
            pl.BlockSpec((1, 64), lambda i: (0, 0)),
        ],
        out_specs=[
            pl.BlockSpec((ROW_BLK, D), lambda i: (i, 0)),
            pl.BlockSpec((ROW_BLK, 64), lambda i: (i, 0)),
        ],
        out_shape=[
            jax.ShapeDtypeStruct((N_NODES, D), jnp.float32),
            jax.ShapeDtypeStruct((N_NODES, 64), jnp.float32),
        ],
    )(p0, p1, u, phi_t, phib2d, head_t, headb2d)


def kernel(x, u, edge_index, W_w, W_b, phi_w, phi_b, head_w, head_b):
    h = _h_matmul(x, W_w.T, W_b.reshape(1, D))
    p0, p1 = _segment_sum_sc(edge_index[0], edge_index[1], h)
    h2, y = _out_tc(p0, p1, u, phi_w.T, phi_b.reshape(1, D),
                    head_w.T, head_b.reshape(1, 64))
    return (h2, y)


# speedup vs baseline: 10.9774x; 1.0688x over previous
import functools

import jax
import jax.numpy as jnp
from jax import lax
from jax.experimental import pallas as pl
from jax.experimental.pallas import tpu as pltpu
from jax.experimental.pallas import tpu_sc as plsc

N_NODES = 10000
N_EDGES = 320000
D = 128

CHUNK = 128
NUM_CHUNKS = N_EDGES // CHUNK
NC, NS = 2, 16
NW = NC * NS
BASE_CHUNKS = NUM_CHUNKS // NW
EXTRA = NUM_CHUNKS - BASE_CHUNKS * NW
ROWS_PER_SUB = 624
TAIL_ROWS = N_NODES - NS * ROWS_PER_SUB
ZROWS = 16

ROW_BLK = 1000


def _h_body(x_ref, w_ref, b_ref, o_ref):
    o_ref[...] = (
        jnp.dot(x_ref[...], w_ref[...], preferred_element_type=jnp.float32)
        + b_ref[...]
    )


def _h_matmul(x, w_t, b2d):
    return pl.pallas_call(
        _h_body,
        grid=(N_NODES // ROW_BLK,),
        in_specs=[
            pl.BlockSpec((ROW_BLK, D), lambda i: (i, 0)),
            pl.BlockSpec((D, D), lambda i: (0, 0)),
            pl.BlockSpec((1, D), lambda i: (0, 0)),
        ],
        out_specs=pl.BlockSpec((ROW_BLK, D), lambda i: (i, 0)),
        out_shape=jax.ShapeDtypeStruct((N_NODES, D), jnp.float32),
    )(x, w_t, b2d)


def _sc_body(src_hbm, dst_hbm, h_hbm, out0, out1,
             sidx0, sidx1, sidx2, didx0, didx1, didx2,
             rows0, rows1, rows2, agg,
             lsem0, lsem1, lsem2, gsem0, gsem1, gsem2):
    c = lax.axis_index("c")
    s = lax.axis_index("s")
    wid = s * NC + c

    start = wid * BASE_CHUNKS + jnp.minimum(wid, EXTRA)
    has_extra = wid < EXTRA

    sidx = [sidx0, sidx1, sidx2]
    didx = [didx0, didx1, didx2]
    rows = [rows0, rows1, rows2]
    lsem = [lsem0, lsem1, lsem2]
    gsem = [gsem0, gsem1, gsem2]

    def chunk_off(j):
        cid = jnp.minimum(start + j, NUM_CHUNKS - 1)
        return pl.multiple_of(cid * CHUNK, CHUNK)

    def fire_l(j, b):
        off = chunk_off(j)
        pltpu.async_copy(src_hbm.at[pl.ds(off, CHUNK)], sidx[b], lsem[b])
        pltpu.async_copy(dst_hbm.at[pl.ds(off, CHUNK)], didx[b], lsem[b])

    def wait_l(b):
        pltpu.make_async_copy(src_hbm.at[pl.ds(0, CHUNK)], sidx[b],
                              lsem[b]).wait()
        pltpu.make_async_copy(dst_hbm.at[pl.ds(0, CHUNK)], didx[b],
                              lsem[b]).wait()

    def fire_g(b):
        pltpu.async_copy(h_hbm.at[sidx[b]], rows[b], gsem[b])

    def wait_g(b):
        pltpu.make_async_copy(h_hbm.at[sidx[b]], rows[b], gsem[b]).wait()

    for b in range(3):
        fire_l(b, b)
    wait_l(0)
    fire_g(0)
    wait_l(1)
    fire_g(1)

    zero = jnp.zeros((16,), jnp.float32)

    def zrow(r, carry):
        for cc in range(8):
            rows2[r, pl.ds(cc * 16, 16)] = zero
        return carry

    lax.fori_loop(0, ZROWS, zrow, 0)
    rs = pl.multiple_of(s * ROWS_PER_SUB, 8)
    zsrc = rows2.at[pl.ds(0, ZROWS)]

    def zcopy(b, carry):
        off = pl.multiple_of(rs + b * ZROWS, 8)
        pltpu.sync_copy(zsrc, agg.at[pl.ds(off, ZROWS)])
        return carry

    lax.fori_loop(0, ROWS_PER_SUB // ZROWS, zcopy, 0)

    @pl.when(s == NS - 1)
    def _():
        pltpu.sync_copy(zsrc, agg.at[pl.ds(NS * ROWS_PER_SUB, TAIL_ROWS)])

    plsc.subcore_barrier()

    def pipe_body(i, carry):
        j0 = 3 * i
        for k in range(3):
            g = (k + 2) % 3
            wait_g(k)
            wait_l(g)
            fire_g(g)
            pltpu.sync_copy(rows[k], agg.at[didx[k]], add=True)
            fire_l(j0 + k + 3, k)
        return carry

    lax.fori_loop(0, BASE_CHUNKS // 3, pipe_body, 0)

    wait_g(0)
    wait_g(1)
    wait_l(2)

    @pl.when(has_extra)
    def _():
        pltpu.sync_copy(rows[0], agg.at[didx[0]], add=True)

    plsc.subcore_barrier()

    out = [out0, out1]
    for ci in range(NC):
        @pl.when(c == ci)
        def _(ci=ci):
            pltpu.sync_copy(agg.at[pl.ds(rs, ROWS_PER_SUB)],
                            out[ci].at[pl.ds(rs, ROWS_PER_SUB)])

            @pl.when(s == NS - 1)
            def _():
                pltpu.sync_copy(
                    agg.at[pl.ds(NS * ROWS_PER_SUB, TAIL_ROWS)],
                    out[ci].at[pl.ds(NS * ROWS_PER_SUB, TAIL_ROWS)])


@functools.partial(jax.jit, donate_argnums=())
def _segment_sum_sc(src, dst, h):
    mesh = plsc.VectorSubcoreMesh(core_axis_name="c", subcore_axis_name="s")
    f = pl.kernel(
        _sc_body,
        out_type=(
            jax.ShapeDtypeStruct((N_NODES, D), jnp.float32),
            jax.ShapeDtypeStruct((N_NODES, D), jnp.float32),
        ),
        mesh=mesh,
        scratch_types=[
            pltpu.VMEM((CHUNK,), jnp.int32),
            pltpu.VMEM((CHUNK,), jnp.int32),
            pltpu.VMEM((CHUNK,), jnp.int32),
            pltpu.VMEM((CHUNK,), jnp.int32),
            pltpu.VMEM((CHUNK,), jnp.int32),
            pltpu.VMEM((CHUNK,), jnp.int32),
            pltpu.VMEM((CHUNK, D), jnp.float32),
            pltpu.VMEM((CHUNK, D), jnp.float32),
            pltpu.VMEM((CHUNK, D), jnp.float32),
            pltpu.VMEM_SHARED((N_NODES, D), jnp.float32),
            pltpu.SemaphoreType.DMA,
            pltpu.SemaphoreType.DMA,
            pltpu.SemaphoreType.DMA,
            pltpu.SemaphoreType.DMA,
            pltpu.SemaphoreType.DMA,
            pltpu.SemaphoreType.DMA,
        ],
    )
    return f(src, dst, h)


def _out_body(p0_ref, p1_ref, u_ref, phit_ref, phib_ref, hwt_ref, hb_ref,
              h2_ref, y_ref):
    t = (
        p0_ref[...] + p1_ref[...]
        + jnp.dot(u_ref[...], phit_ref[...], preferred_element_type=jnp.float32)
        + phib_ref[...]
    )
    h2 = jnp.maximum(t, 0.0)
    h2_ref[...] = h2
    y_ref[...] = (
        jnp.dot(h2, hwt_ref[...], preferred_element_type=jnp.float32)
        + hb_ref[...]
    )


def _out_tc(p0, p1, u, phi_t, phib2d, head_t, headb2d):
    return pl.pallas_call(
        _out_body,
        grid=(N_NODES // ROW_BLK,),
        in_specs=[
            pl.BlockSpec((ROW_BLK, D), lambda i: (i, 0)),
            pl.BlockSpec((ROW_BLK, D), lambda i: (i, 0)),
            pl.BlockSpec((ROW_BLK, D), lambda i: (i, 0)),
            pl.BlockSpec((D, D), lambda i: (0, 0)),
            pl.BlockSpec((1, D), lambda i: (0, 0)),
            pl.BlockSpec((D, 64), lambda i: (0, 0)),
            pl.BlockSpec((1, 64), lambda i: (0, 0)),
        ],
        out_specs=[
            pl.BlockSpec((ROW_BLK, D), lambda i: (i, 0)),
            pl.BlockSpec((ROW_BLK, 64), lambda i: (i, 0)),
        ],
        out_shape=[
            jax.ShapeDtypeStruct((N_NODES, D), jnp.float32),
            jax.ShapeDtypeStruct((N_NODES, 64), jnp.float32),
        ],
    )(p0, p1, u, phi_t, phib2d, head_t, headb2d)


def kernel(x, u, edge_index, W_w, W_b, phi_w, phi_b, head_w, head_b):
    h = _h_matmul(x, W_w.T, W_b.reshape(1, D))
    p0, p1 = _segment_sum_sc(edge_index[0], edge_index[1], h)
    h2, y = _out_tc(p0, p1, u, phi_w.T, phi_b.reshape(1, D),
                    head_w.T, head_b.reshape(1, 64))
    return (h2, y)
